# baseline (speedup 1.0000x reference)
import functools

import jax
import jax.numpy as jnp
from jax import lax
from jax.experimental import pallas as pl
from jax.experimental.pallas import tpu as pltpu
from jax.experimental.pallas import tpu_sc as plsc

_H = 32
_W = 32
_D = 384
_N_BOXES = 5000
_NSC = 1024
_NTC = _N_BOXES - _NSC
_TCB = 256
_NTC_PAD = -(-_NTC // _TCB) * _TCB
_NB = _NSC
_N_TILES = 32
_BPW = _NB // _N_TILES
_CH = 16
_NCH = _BPW // _CH
_LANES = 16
_SW = 40
_SROWS = (_H + 1) * _SW
_TAIL = 8


def _integral_kernel(feat_ref, s_ref, rc_ref):
    acc = feat_ref[0]
    rc_ref[0] = acc
    for i in range(1, _H):
        acc = acc + feat_ref[i]
        rc_ref[i] = acc
    s_ref[0] = jnp.zeros((_SW, _D), jnp.float32)
    cacc = jnp.zeros((_H, _D), jnp.float32)
    s_ref[1:_H + 1, 0, :] = cacc
    for w in range(_W):
        cacc = cacc + rc_ref[:, w, :]
        s_ref[1:_H + 1, w + 1, :] = cacc


def _integral_image(feat_map):
    return pl.pallas_call(
        _integral_kernel,
        out_shape=jax.ShapeDtypeStruct((_H + 1, _SW, _D), jnp.float32),
        scratch_shapes=[pltpu.VMEM((_H, _W, _D), jnp.float32)],
    )(feat_map)


def _round_half_even_nonneg(t):
    i = t.astype(jnp.int32)
    f = t - i.astype(jnp.float32)
    up = (f > 0.5) | ((f == 0.5) & ((i & 1) == 1))
    return jnp.where(up, i + 1, i)


def _sc_body(s_hbm, x1_hbm, y1_hbm, x2_hbm, y2_hbm, out_hbm,
             x1_v, y1_v, x2_v, y2_v, idx_a, idx_b, rows_a, rows_b,
             out_a, out_b, inv_v, gsem_a, gsem_b, osem_a, osem_b, bsem):
    wid = lax.axis_index("s") * 2 + lax.axis_index("c")
    base = wid * _BPW
    pltpu.async_copy(x1_hbm.at[pl.ds(base, _BPW)], x1_v, bsem)
    pltpu.async_copy(y1_hbm.at[pl.ds(base, _BPW)], y1_v, bsem)
    pltpu.async_copy(x2_hbm.at[pl.ds(base, _BPW)], x2_v, bsem)
    pltpu.async_copy(y2_hbm.at[pl.ds(base, _BPW)], y2_v, bsem)
    for v in (x1_v, y1_v, x2_v, y2_v):
        pltpu.make_async_copy(x1_hbm.at[pl.ds(base, _BPW)], v, bsem).wait()

    def fill_idx(ci, idx_r):
        off = ci * _CH
        x1 = x1_v[pl.ds(off, _LANES)]
        y1 = y1_v[pl.ds(off, _LANES)]
        x2 = x2_v[pl.ds(off, _LANES)]
        y2 = y2_v[pl.ds(off, _LANES)]
        zero = jnp.zeros((_LANES,), jnp.int32)
        wvec = jnp.full((_LANES,), _W, jnp.int32)
        hvec = jnp.full((_LANES,), _H, jnp.int32)
        cl = jnp.maximum(zero, (x1 * float(_W)).astype(jnp.int32))
        ch = jnp.minimum(wvec, jnp.maximum(
            cl + 1, _round_half_even_nonneg(x2 * float(_W) + 0.5)))
        rl = jnp.maximum(zero, (y1 * float(_H)).astype(jnp.int32))
        rh = jnp.minimum(hvec, jnp.maximum(
            rl + 1, _round_half_even_nonneg(y2 * float(_H) + 0.5)))
        stride = _SW
        idx_r[pl.ds(0, _LANES)] = rl * stride + cl
        idx_r[pl.ds(16, _LANES)] = rl * stride + ch
        idx_r[pl.ds(32, _LANES)] = rh * stride + cl
        idx_r[pl.ds(48, _LANES)] = rh * stride + ch
        cnt = (rh - rl) * (ch - cl)
        inv_v[pl.ds(off, _LANES)] = 1.0 / cnt.astype(jnp.float32)

    def combine(ci, rows_r, out_r):
        iv = inv_v[pl.ds(ci * _CH, _LANES)]

        def box(b, carry):
            ib = iv.at[jnp.full((_LANES,), b, jnp.int32)].get(
                mode="promise_in_bounds")
            for d in range(_D // _LANES):
                ds_ = pl.ds(d * _LANES, _LANES)
                out_r[b, ds_] = (rows_r[48 + b, ds_] - rows_r[16 + b, ds_]
                                 - rows_r[32 + b, ds_] + rows_r[b, ds_]) * ib
            return carry

        lax.fori_loop(0, _CH, box, None)

    fill_idx(0, idx_a)
    pltpu.async_copy(s_hbm.at[idx_a], rows_a, gsem_a)
    fill_idx(1, idx_b)
    pltpu.async_copy(s_hbm.at[idx_b], rows_b, gsem_b)

    def out_issue(ci, out_r, osem):
        start = base + ci * _CH
        full = start + _CH <= _NB
        part = (start < _NB) & (start + _CH > _NB)

        @pl.when(full)
        def _():
            pltpu.async_copy(out_r, out_hbm.at[pl.ds(start, _CH)], osem)

        @pl.when(part)
        def _():
            pltpu.async_copy(out_r.at[pl.ds(0, _TAIL)],
                             out_hbm.at[pl.ds(start, _TAIL)], osem)

    def out_wait(ci, out_r, osem):
        start = base + ci * _CH
        full = start + _CH <= _NB
        part = (start < _NB) & (start + _CH > _NB)

        @pl.when(full)
        def _():
            pltpu.make_async_copy(
                out_r, out_hbm.at[pl.ds(base, _CH)], osem).wait()

        @pl.when(part)
        def _():
            pltpu.make_async_copy(out_r.at[pl.ds(0, _TAIL)],
                                  out_hbm.at[pl.ds(base, _TAIL)], osem).wait()

    def stage(g, ci, idx_r, rows_r, out_r, gsem, osem):
        pltpu.make_async_copy(s_hbm.at[idx_r], rows_r, gsem).wait()

        @pl.when(g > 0)
        def _():
            out_wait(ci - 2, out_r, osem)

        combine(ci, rows_r, out_r)
        out_issue(ci, out_r, osem)

        @pl.when(ci + 2 < _NCH)
        def _():
            fill_idx(ci + 2, idx_r)
            pltpu.async_copy(s_hbm.at[idx_r], rows_r, gsem)

    def pair(g, carry):
        stage(g, 2 * g, idx_a, rows_a, out_a, gsem_a, osem_a)
        stage(g, 2 * g + 1, idx_b, rows_b, out_b, gsem_b, osem_b)
        return carry

    lax.fori_loop(0, _NCH // 2, pair, None)
    out_wait(_NCH - 2, out_a, osem_a)
    out_wait(_NCH - 1, out_b, osem_b)


def _roi_pool_sc(s_flat, x1, y1, x2, y2):
    mesh = plsc.VectorSubcoreMesh(core_axis_name="c", subcore_axis_name="s")
    f = functools.partial(
        pl.kernel,
        out_type=jax.ShapeDtypeStruct((_NB, _D), jnp.float32),
        mesh=mesh,
        scratch_types=[
            pltpu.VMEM((_BPW,), jnp.float32),
            pltpu.VMEM((_BPW,), jnp.float32),
            pltpu.VMEM((_BPW,), jnp.float32),
            pltpu.VMEM((_BPW,), jnp.float32),
            pltpu.VMEM((4 * _CH,), jnp.int32),
            pltpu.VMEM((4 * _CH,), jnp.int32),
            pltpu.VMEM((4 * _CH, _D), jnp.float32),
            pltpu.VMEM((4 * _CH, _D), jnp.float32),
            pltpu.VMEM((_CH, _D), jnp.float32),
            pltpu.VMEM((_CH, _D), jnp.float32),
            pltpu.VMEM((_BPW,), jnp.float32),
            pltpu.SemaphoreType.DMA,
            pltpu.SemaphoreType.DMA,
            pltpu.SemaphoreType.DMA,
            pltpu.SemaphoreType.DMA,
            pltpu.SemaphoreType.DMA,
        ],
    )(_sc_body)
    return f(s_flat, x1, y1, x2, y2)


def _tc_einsum_kernel(x1_ref, y1_ref, x2_ref, y2_ref, hi_ref, lo_ref, out_ref):
    x1 = x1_ref[...]
    y1 = y1_ref[...]
    x2 = x2_ref[...]
    y2 = y2_ref[...]
    cl = jnp.maximum(0, (x1 * float(_W)).astype(jnp.int32))
    ch = jnp.minimum(_W, jnp.maximum(
        cl + 1, _round_half_even_nonneg(x2 * float(_W) + 0.5)))
    rl = jnp.maximum(0, (y1 * float(_H)).astype(jnp.int32))
    rh = jnp.minimum(_H, jnp.maximum(
        rl + 1, _round_half_even_nonneg(y2 * float(_H) + 0.5)))
    one = jnp.float32(1.0)
    zero = jnp.float32(0.0)
    ih = lax.broadcasted_iota(jnp.int32, (_TCB, _H), 1)
    rowm = jnp.where((ih >= rl[:, None]) & (ih < rh[:, None]), one, zero)
    iw = lax.broadcasted_iota(jnp.int32, (_TCB, _W), 1)
    colm = jnp.where((iw >= cl[:, None]) & (iw < ch[:, None]), one, zero)
    jr = lax.broadcasted_iota(jnp.int32, (_H, _H * _W), 0)
    jc = lax.broadcasted_iota(jnp.int32, (_H, _H * _W), 1)
    eh = jnp.where(jr == (jc // _W), one, zero)
    ew = jnp.where(jr == (jc % _W), one, zero)
    rexp = jnp.dot(rowm, eh, preferred_element_type=jnp.float32)
    cexp = jnp.dot(colm, ew, preferred_element_type=jnp.float32)
    mask = (rexp * cexp).astype(jnp.bfloat16)
    acc = (jnp.dot(mask, hi_ref[...], preferred_element_type=jnp.float32)
           + jnp.dot(mask, lo_ref[...], preferred_element_type=jnp.float32))
    cnt = ((rh - rl) * (ch - cl)).astype(jnp.float32)
    out_ref[...] = acc * (1.0 / cnt)[:, None]


def _roi_pool_tc(x1, y1, x2, y2, hi, lo):
    grid = _NTC_PAD // _TCB
    return pl.pallas_call(
        _tc_einsum_kernel,
        grid=(grid,),
        in_specs=[
            pl.BlockSpec((_TCB,), lambda i: (i,)),
            pl.BlockSpec((_TCB,), lambda i: (i,)),
            pl.BlockSpec((_TCB,), lambda i: (i,)),
            pl.BlockSpec((_TCB,), lambda i: (i,)),
            pl.BlockSpec((_H * _W, _D), lambda i: (0, 0)),
            pl.BlockSpec((_H * _W, _D), lambda i: (0, 0)),
        ],
        out_specs=pl.BlockSpec((_TCB, _D), lambda i: (i, 0)),
        out_shape=jax.ShapeDtypeStruct((_NTC, _D), jnp.float32),
    )(x1, y1, x2, y2, hi, lo)


def kernel(feat_map, boxes):
    s_flat = _integral_image(feat_map).reshape(_SROWS, _D)
    bp = jnp.zeros((_NTC_PAD + _NSC, 4), jnp.float32).at[:_N_BOXES].set(boxes)
    feat2 = feat_map.reshape(_H * _W, _D)
    hi = feat2.astype(jnp.bfloat16)
    lo = (feat2 - hi.astype(jnp.float32)).astype(jnp.bfloat16)
    out_sc = _roi_pool_sc(s_flat, bp[_NTC:_NTC + _NSC, 0],
                          bp[_NTC:_NTC + _NSC, 1],
                          bp[_NTC:_NTC + _NSC, 2],
                          bp[_NTC:_NTC + _NSC, 3])
    out_tc = _roi_pool_tc(bp[:_NTC_PAD, 0], bp[:_NTC_PAD, 1],
                          bp[:_NTC_PAD, 2], bp[:_NTC_PAD, 3], hi, lo)
    return jnp.concatenate([out_tc, out_sc], axis=0)


# speedup vs baseline: 1.0183x; 1.0183x over previous
import functools

import jax
import jax.numpy as jnp
from jax import lax
from jax.experimental import pallas as pl
from jax.experimental.pallas import tpu as pltpu
from jax.experimental.pallas import tpu_sc as plsc

_H = 32
_W = 32
_D = 384
_N_BOXES = 5000
_NSC = 1024
_NTC = _N_BOXES - _NSC
_TCB = 256
_NTC_PAD = -(-_NTC // _TCB) * _TCB
_NB = _NSC
_N_TILES = 32
_BPW = _NB // _N_TILES
_CH = 16
_NCH = _BPW // _CH
_LANES = 16
_SW = 40
_SROWS = (_H + 1) * _SW
_TAIL = 8


def _integral_kernel(feat_ref, s_ref, rc_ref):
    acc = feat_ref[0]
    rc_ref[0] = acc
    for i in range(1, _H):
        acc = acc + feat_ref[i]
        rc_ref[i] = acc
    s_ref[0] = jnp.zeros((_SW, _D), jnp.float32)
    cacc = jnp.zeros((_H, _D), jnp.float32)
    s_ref[1:_H + 1, 0, :] = cacc
    for w in range(_W):
        cacc = cacc + rc_ref[:, w, :]
        s_ref[1:_H + 1, w + 1, :] = cacc


def _integral_image(feat_map):
    return pl.pallas_call(
        _integral_kernel,
        out_shape=jax.ShapeDtypeStruct((_H + 1, _SW, _D), jnp.float32),
        scratch_shapes=[pltpu.VMEM((_H, _W, _D), jnp.float32)],
    )(feat_map)


def _round_half_even_nonneg(t):
    i = t.astype(jnp.int32)
    f = t - i.astype(jnp.float32)
    up = (f > 0.5) | ((f == 0.5) & ((i & 1) == 1))
    return jnp.where(up, i + 1, i)


def _sc_body(s_hbm, x1_hbm, y1_hbm, x2_hbm, y2_hbm, out_hbm,
             x1_v, y1_v, x2_v, y2_v, idx_a, idx_b, rows_a, rows_b,
             out_a, out_b, inv_v, gsem_a, gsem_b, osem_a, osem_b, bsem):
    wid = lax.axis_index("s") * 2 + lax.axis_index("c")
    base = wid * _BPW
    pltpu.async_copy(x1_hbm.at[pl.ds(base, _BPW)], x1_v, bsem)
    pltpu.async_copy(y1_hbm.at[pl.ds(base, _BPW)], y1_v, bsem)
    pltpu.async_copy(x2_hbm.at[pl.ds(base, _BPW)], x2_v, bsem)
    pltpu.async_copy(y2_hbm.at[pl.ds(base, _BPW)], y2_v, bsem)
    for v in (x1_v, y1_v, x2_v, y2_v):
        pltpu.make_async_copy(x1_hbm.at[pl.ds(base, _BPW)], v, bsem).wait()

    def fill_idx(ci, idx_r):
        off = ci * _CH
        x1 = x1_v[pl.ds(off, _LANES)]
        y1 = y1_v[pl.ds(off, _LANES)]
        x2 = x2_v[pl.ds(off, _LANES)]
        y2 = y2_v[pl.ds(off, _LANES)]
        zero = jnp.zeros((_LANES,), jnp.int32)
        wvec = jnp.full((_LANES,), _W, jnp.int32)
        hvec = jnp.full((_LANES,), _H, jnp.int32)
        cl = jnp.maximum(zero, (x1 * float(_W)).astype(jnp.int32))
        ch = jnp.minimum(wvec, jnp.maximum(
            cl + 1, _round_half_even_nonneg(x2 * float(_W) + 0.5)))
        rl = jnp.maximum(zero, (y1 * float(_H)).astype(jnp.int32))
        rh = jnp.minimum(hvec, jnp.maximum(
            rl + 1, _round_half_even_nonneg(y2 * float(_H) + 0.5)))
        stride = _SW
        idx_r[pl.ds(0, _LANES)] = rl * stride + cl
        idx_r[pl.ds(16, _LANES)] = rl * stride + ch
        idx_r[pl.ds(32, _LANES)] = rh * stride + cl
        idx_r[pl.ds(48, _LANES)] = rh * stride + ch
        cnt = (rh - rl) * (ch - cl)
        inv_v[pl.ds(off, _LANES)] = 1.0 / cnt.astype(jnp.float32)

    def combine(ci, rows_r, out_r):
        iv = inv_v[pl.ds(ci * _CH, _LANES)]

        def box(b, carry):
            ib = iv.at[jnp.full((_LANES,), b, jnp.int32)].get(
                mode="promise_in_bounds")
            for d in range(_D // _LANES):
                ds_ = pl.ds(d * _LANES, _LANES)
                out_r[b, ds_] = (rows_r[48 + b, ds_] - rows_r[16 + b, ds_]
                                 - rows_r[32 + b, ds_] + rows_r[b, ds_]) * ib
            return carry

        lax.fori_loop(0, _CH, box, None)

    fill_idx(0, idx_a)
    pltpu.async_copy(s_hbm.at[idx_a], rows_a, gsem_a)
    fill_idx(1, idx_b)
    pltpu.async_copy(s_hbm.at[idx_b], rows_b, gsem_b)

    def out_issue(ci, out_r, osem):
        start = base + ci * _CH
        full = start + _CH <= _NB
        part = (start < _NB) & (start + _CH > _NB)

        @pl.when(full)
        def _():
            pltpu.async_copy(out_r, out_hbm.at[pl.ds(start, _CH)], osem)

        @pl.when(part)
        def _():
            pltpu.async_copy(out_r.at[pl.ds(0, _TAIL)],
                             out_hbm.at[pl.ds(start, _TAIL)], osem)

    def out_wait(ci, out_r, osem):
        start = base + ci * _CH
        full = start + _CH <= _NB
        part = (start < _NB) & (start + _CH > _NB)

        @pl.when(full)
        def _():
            pltpu.make_async_copy(
                out_r, out_hbm.at[pl.ds(base, _CH)], osem).wait()

        @pl.when(part)
        def _():
            pltpu.make_async_copy(out_r.at[pl.ds(0, _TAIL)],
                                  out_hbm.at[pl.ds(base, _TAIL)], osem).wait()

    def stage(g, ci, idx_r, rows_r, out_r, gsem, osem):
        pltpu.make_async_copy(s_hbm.at[idx_r], rows_r, gsem).wait()

        @pl.when(g > 0)
        def _():
            out_wait(ci - 2, out_r, osem)

        combine(ci, rows_r, out_r)
        out_issue(ci, out_r, osem)

        @pl.when(ci + 2 < _NCH)
        def _():
            fill_idx(ci + 2, idx_r)
            pltpu.async_copy(s_hbm.at[idx_r], rows_r, gsem)

    def pair(g, carry):
        stage(g, 2 * g, idx_a, rows_a, out_a, gsem_a, osem_a)
        stage(g, 2 * g + 1, idx_b, rows_b, out_b, gsem_b, osem_b)
        return carry

    lax.fori_loop(0, _NCH // 2, pair, None)
    out_wait(_NCH - 2, out_a, osem_a)
    out_wait(_NCH - 1, out_b, osem_b)


def _roi_pool_sc(s_flat, x1, y1, x2, y2):
    mesh = plsc.VectorSubcoreMesh(core_axis_name="c", subcore_axis_name="s")
    f = functools.partial(
        pl.kernel,
        out_type=jax.ShapeDtypeStruct((_NB, _D), jnp.float32),
        mesh=mesh,
        scratch_types=[
            pltpu.VMEM((_BPW,), jnp.float32),
            pltpu.VMEM((_BPW,), jnp.float32),
            pltpu.VMEM((_BPW,), jnp.float32),
            pltpu.VMEM((_BPW,), jnp.float32),
            pltpu.VMEM((4 * _CH,), jnp.int32),
            pltpu.VMEM((4 * _CH,), jnp.int32),
            pltpu.VMEM((4 * _CH, _D), jnp.float32),
            pltpu.VMEM((4 * _CH, _D), jnp.float32),
            pltpu.VMEM((_CH, _D), jnp.float32),
            pltpu.VMEM((_CH, _D), jnp.float32),
            pltpu.VMEM((_BPW,), jnp.float32),
            pltpu.SemaphoreType.DMA,
            pltpu.SemaphoreType.DMA,
            pltpu.SemaphoreType.DMA,
            pltpu.SemaphoreType.DMA,
            pltpu.SemaphoreType.DMA,
        ],
    )(_sc_body)
    return f(s_flat, x1, y1, x2, y2)


def _tc_einsum_kernel(x1_ref, y1_ref, x2_ref, y2_ref, hi_ref, lo_ref, out_ref):
    x1 = x1_ref[...]
    y1 = y1_ref[...]
    x2 = x2_ref[...]
    y2 = y2_ref[...]
    cl = jnp.maximum(0, (x1 * float(_W)).astype(jnp.int32))
    ch = jnp.minimum(_W, jnp.maximum(
        cl + 1, _round_half_even_nonneg(x2 * float(_W) + 0.5)))
    rl = jnp.maximum(0, (y1 * float(_H)).astype(jnp.int32))
    rh = jnp.minimum(_H, jnp.maximum(
        rl + 1, _round_half_even_nonneg(y2 * float(_H) + 0.5)))
    one = jnp.float32(1.0)
    zero = jnp.float32(0.0)
    jv = lax.broadcasted_iota(jnp.int32, (_TCB, _H * _W), 1)
    jm = jv & (_W - 1)
    cond = ((jv >= (rl * _W)[:, None]) & (jv < (rh * _W)[:, None])
            & (jm >= cl[:, None]) & (jm < ch[:, None]))
    mask = jnp.where(cond, one, zero).astype(jnp.bfloat16)
    acc = (jnp.dot(mask, hi_ref[...], preferred_element_type=jnp.float32)
           + jnp.dot(mask, lo_ref[...], preferred_element_type=jnp.float32))
    cnt = ((rh - rl) * (ch - cl)).astype(jnp.float32)
    out_ref[...] = acc * (1.0 / cnt)[:, None]


def _roi_pool_tc(x1, y1, x2, y2, hi, lo):
    grid = _NTC_PAD // _TCB
    return pl.pallas_call(
        _tc_einsum_kernel,
        grid=(grid,),
        in_specs=[
            pl.BlockSpec((_TCB,), lambda i: (i,)),
            pl.BlockSpec((_TCB,), lambda i: (i,)),
            pl.BlockSpec((_TCB,), lambda i: (i,)),
            pl.BlockSpec((_TCB,), lambda i: (i,)),
            pl.BlockSpec((_H * _W, _D), lambda i: (0, 0)),
            pl.BlockSpec((_H * _W, _D), lambda i: (0, 0)),
        ],
        out_specs=pl.BlockSpec((_TCB, _D), lambda i: (i, 0)),
        out_shape=jax.ShapeDtypeStruct((_NTC, _D), jnp.float32),
    )(x1, y1, x2, y2, hi, lo)


def kernel(feat_map, boxes):
    s_flat = _integral_image(feat_map).reshape(_SROWS, _D)
    bp = jnp.zeros((_NTC_PAD + _NSC, 4), jnp.float32).at[:_N_BOXES].set(boxes)
    feat2 = feat_map.reshape(_H * _W, _D)
    hi = feat2.astype(jnp.bfloat16)
    lo = (feat2 - hi.astype(jnp.float32)).astype(jnp.bfloat16)
    out_sc = _roi_pool_sc(s_flat, bp[_NTC:_NTC + _NSC, 0],
                          bp[_NTC:_NTC + _NSC, 1],
                          bp[_NTC:_NTC + _NSC, 2],
                          bp[_NTC:_NTC + _NSC, 3])
    out_tc = _roi_pool_tc(bp[:_NTC_PAD, 0], bp[:_NTC_PAD, 1],
                          bp[:_NTC_PAD, 2], bp[:_NTC_PAD, 3], hi, lo)
    return jnp.concatenate([out_tc, out_sc], axis=0)


# speedup vs baseline: 1.1804x; 1.1591x over previous
import functools

import jax
import jax.numpy as jnp
from jax import lax
from jax.experimental import pallas as pl
from jax.experimental.pallas import tpu as pltpu
from jax.experimental.pallas import tpu_sc as plsc

_H = 32
_W = 32
_D = 384
_N_BOXES = 5000
_NSC = 2048
_NTC = _N_BOXES - _NSC
_TCB = 256
_NTC_PAD = -(-_NTC // _TCB) * _TCB
_NB = _NSC
_N_TILES = 32
_BPW = _NB // _N_TILES
_CH = 16
_NCH = _BPW // _CH
_LANES = 16
_SW = 40
_SROWS = (_H + 1) * _SW
_TAIL = 8


def _integral_kernel(feat_ref, s_ref, rc_ref):
    acc = feat_ref[0]
    rc_ref[0] = acc
    for i in range(1, _H):
        acc = acc + feat_ref[i]
        rc_ref[i] = acc
    s_ref[0] = jnp.zeros((_SW, _D), jnp.float32)
    cacc = jnp.zeros((_H, _D), jnp.float32)
    s_ref[1:_H + 1, 0, :] = cacc
    for w in range(_W):
        cacc = cacc + rc_ref[:, w, :]
        s_ref[1:_H + 1, w + 1, :] = cacc


def _integral_image(feat_map):
    return pl.pallas_call(
        _integral_kernel,
        out_shape=jax.ShapeDtypeStruct((_H + 1, _SW, _D), jnp.float32),
        scratch_shapes=[pltpu.VMEM((_H, _W, _D), jnp.float32)],
    )(feat_map)


def _round_half_even_nonneg(t):
    i = t.astype(jnp.int32)
    f = t - i.astype(jnp.float32)
    up = (f > 0.5) | ((f == 0.5) & ((i & 1) == 1))
    return jnp.where(up, i + 1, i)


def _sc_body(s_hbm, x1_hbm, y1_hbm, x2_hbm, y2_hbm, out_hbm,
             x1_v, y1_v, x2_v, y2_v, idx_a, idx_b, rows_a, rows_b,
             out_a, out_b, inv_v, gsem_a, gsem_b, osem_a, osem_b, bsem):
    wid = lax.axis_index("s") * 2 + lax.axis_index("c")
    base = wid * _BPW
    pltpu.async_copy(x1_hbm.at[pl.ds(base, _BPW)], x1_v, bsem)
    pltpu.async_copy(y1_hbm.at[pl.ds(base, _BPW)], y1_v, bsem)
    pltpu.async_copy(x2_hbm.at[pl.ds(base, _BPW)], x2_v, bsem)
    pltpu.async_copy(y2_hbm.at[pl.ds(base, _BPW)], y2_v, bsem)
    for v in (x1_v, y1_v, x2_v, y2_v):
        pltpu.make_async_copy(x1_hbm.at[pl.ds(base, _BPW)], v, bsem).wait()

    def fill_idx(ci, idx_r):
        off = ci * _CH
        x1 = x1_v[pl.ds(off, _LANES)]
        y1 = y1_v[pl.ds(off, _LANES)]
        x2 = x2_v[pl.ds(off, _LANES)]
        y2 = y2_v[pl.ds(off, _LANES)]
        zero = jnp.zeros((_LANES,), jnp.int32)
        wvec = jnp.full((_LANES,), _W, jnp.int32)
        hvec = jnp.full((_LANES,), _H, jnp.int32)
        cl = jnp.maximum(zero, (x1 * float(_W)).astype(jnp.int32))
        ch = jnp.minimum(wvec, jnp.maximum(
            cl + 1, _round_half_even_nonneg(x2 * float(_W) + 0.5)))
        rl = jnp.maximum(zero, (y1 * float(_H)).astype(jnp.int32))
        rh = jnp.minimum(hvec, jnp.maximum(
            rl + 1, _round_half_even_nonneg(y2 * float(_H) + 0.5)))
        stride = _SW
        idx_r[pl.ds(0, _LANES)] = rl * stride + cl
        idx_r[pl.ds(16, _LANES)] = rl * stride + ch
        idx_r[pl.ds(32, _LANES)] = rh * stride + cl
        idx_r[pl.ds(48, _LANES)] = rh * stride + ch
        cnt = (rh - rl) * (ch - cl)
        inv_v[pl.ds(off, _LANES)] = 1.0 / cnt.astype(jnp.float32)

    def combine(ci, rows_r, out_r):
        iv = inv_v[pl.ds(ci * _CH, _LANES)]

        def box(b, carry):
            ib = iv.at[jnp.full((_LANES,), b, jnp.int32)].get(
                mode="promise_in_bounds")
            for d in range(_D // _LANES):
                ds_ = pl.ds(d * _LANES, _LANES)
                out_r[b, ds_] = (rows_r[48 + b, ds_] - rows_r[16 + b, ds_]
                                 - rows_r[32 + b, ds_] + rows_r[b, ds_]) * ib
            return carry

        lax.fori_loop(0, _CH, box, None)

    fill_idx(0, idx_a)
    pltpu.async_copy(s_hbm.at[idx_a], rows_a, gsem_a)
    fill_idx(1, idx_b)
    pltpu.async_copy(s_hbm.at[idx_b], rows_b, gsem_b)

    def out_issue(ci, out_r, osem):
        start = base + ci * _CH
        full = start + _CH <= _NB
        part = (start < _NB) & (start + _CH > _NB)

        @pl.when(full)
        def _():
            pltpu.async_copy(out_r, out_hbm.at[pl.ds(start, _CH)], osem)

        @pl.when(part)
        def _():
            pltpu.async_copy(out_r.at[pl.ds(0, _TAIL)],
                             out_hbm.at[pl.ds(start, _TAIL)], osem)

    def out_wait(ci, out_r, osem):
        start = base + ci * _CH
        full = start + _CH <= _NB
        part = (start < _NB) & (start + _CH > _NB)

        @pl.when(full)
        def _():
            pltpu.make_async_copy(
                out_r, out_hbm.at[pl.ds(base, _CH)], osem).wait()

        @pl.when(part)
        def _():
            pltpu.make_async_copy(out_r.at[pl.ds(0, _TAIL)],
                                  out_hbm.at[pl.ds(base, _TAIL)], osem).wait()

    def stage(g, ci, idx_r, rows_r, out_r, gsem, osem):
        pltpu.make_async_copy(s_hbm.at[idx_r], rows_r, gsem).wait()

        @pl.when(g > 0)
        def _():
            out_wait(ci - 2, out_r, osem)

        combine(ci, rows_r, out_r)
        out_issue(ci, out_r, osem)

        @pl.when(ci + 2 < _NCH)
        def _():
            fill_idx(ci + 2, idx_r)
            pltpu.async_copy(s_hbm.at[idx_r], rows_r, gsem)

    def pair(g, carry):
        stage(g, 2 * g, idx_a, rows_a, out_a, gsem_a, osem_a)
        stage(g, 2 * g + 1, idx_b, rows_b, out_b, gsem_b, osem_b)
        return carry

    lax.fori_loop(0, _NCH // 2, pair, None)
    out_wait(_NCH - 2, out_a, osem_a)
    out_wait(_NCH - 1, out_b, osem_b)


def _roi_pool_sc(s_flat, x1, y1, x2, y2):
    mesh = plsc.VectorSubcoreMesh(core_axis_name="c", subcore_axis_name="s")
    f = functools.partial(
        pl.kernel,
        out_type=jax.ShapeDtypeStruct((_NB, _D), jnp.float32),
        mesh=mesh,
        scratch_types=[
            pltpu.VMEM((_BPW,), jnp.float32),
            pltpu.VMEM((_BPW,), jnp.float32),
            pltpu.VMEM((_BPW,), jnp.float32),
            pltpu.VMEM((_BPW,), jnp.float32),
            pltpu.VMEM((4 * _CH,), jnp.int32),
            pltpu.VMEM((4 * _CH,), jnp.int32),
            pltpu.VMEM((4 * _CH, _D), jnp.float32),
            pltpu.VMEM((4 * _CH, _D), jnp.float32),
            pltpu.VMEM((_CH, _D), jnp.float32),
            pltpu.VMEM((_CH, _D), jnp.float32),
            pltpu.VMEM((_BPW,), jnp.float32),
            pltpu.SemaphoreType.DMA,
            pltpu.SemaphoreType.DMA,
            pltpu.SemaphoreType.DMA,
            pltpu.SemaphoreType.DMA,
            pltpu.SemaphoreType.DMA,
        ],
    )(_sc_body)
    return f(s_flat, x1, y1, x2, y2)


def _tc_einsum_kernel(x1_ref, y1_ref, x2_ref, y2_ref, hi_ref, lo_ref, out_ref):
    x1 = x1_ref[...]
    y1 = y1_ref[...]
    x2 = x2_ref[...]
    y2 = y2_ref[...]
    cl = jnp.maximum(0, (x1 * float(_W)).astype(jnp.int32))
    ch = jnp.minimum(_W, jnp.maximum(
        cl + 1, _round_half_even_nonneg(x2 * float(_W) + 0.5)))
    rl = jnp.maximum(0, (y1 * float(_H)).astype(jnp.int32))
    rh = jnp.minimum(_H, jnp.maximum(
        rl + 1, _round_half_even_nonneg(y2 * float(_H) + 0.5)))
    one = jnp.float32(1.0)
    zero = jnp.float32(0.0)
    jv = lax.broadcasted_iota(jnp.int32, (_TCB, _H * _W), 1)
    jm = jv & (_W - 1)
    cond = ((jv >= (rl * _W)[:, None]) & (jv < (rh * _W)[:, None])
            & (jm >= cl[:, None]) & (jm < ch[:, None]))
    mask = jnp.where(cond, one, zero).astype(jnp.bfloat16)
    acc = (jnp.dot(mask, hi_ref[...], preferred_element_type=jnp.float32)
           + jnp.dot(mask, lo_ref[...], preferred_element_type=jnp.float32))
    cnt = ((rh - rl) * (ch - cl)).astype(jnp.float32)
    out_ref[...] = acc * (1.0 / cnt)[:, None]


def _roi_pool_tc(x1, y1, x2, y2, hi, lo):
    grid = _NTC_PAD // _TCB
    return pl.pallas_call(
        _tc_einsum_kernel,
        grid=(grid,),
        in_specs=[
            pl.BlockSpec((_TCB,), lambda i: (i,)),
            pl.BlockSpec((_TCB,), lambda i: (i,)),
            pl.BlockSpec((_TCB,), lambda i: (i,)),
            pl.BlockSpec((_TCB,), lambda i: (i,)),
            pl.BlockSpec((_H * _W, _D), lambda i: (0, 0)),
            pl.BlockSpec((_H * _W, _D), lambda i: (0, 0)),
        ],
        out_specs=pl.BlockSpec((_TCB, _D), lambda i: (i, 0)),
        out_shape=jax.ShapeDtypeStruct((_N_BOXES, _D), jnp.float32),
    )(x1, y1, x2, y2, hi, lo)


def kernel(feat_map, boxes):
    s_flat = _integral_image(feat_map).reshape(_SROWS, _D)
    bp = jnp.zeros((_NTC_PAD + _NSC, 4), jnp.float32).at[:_N_BOXES].set(boxes)
    feat2 = feat_map.reshape(_H * _W, _D)
    hi = feat2.astype(jnp.bfloat16)
    lo = (feat2 - hi.astype(jnp.float32)).astype(jnp.bfloat16)
    out_sc = _roi_pool_sc(s_flat, bp[_NTC:_NTC + _NSC, 0],
                          bp[_NTC:_NTC + _NSC, 1],
                          bp[_NTC:_NTC + _NSC, 2],
                          bp[_NTC:_NTC + _NSC, 3])
    out_tc = _roi_pool_tc(bp[:_NTC_PAD, 0], bp[:_NTC_PAD, 1],
                          bp[:_NTC_PAD, 2], bp[:_NTC_PAD, 3], hi, lo)
    return lax.dynamic_update_slice(out_tc, out_sc, (_NTC, 0))


# speedup vs baseline: 1.1901x; 1.0082x over previous
import functools

import jax
import jax.numpy as jnp
from jax import lax
from jax.experimental import pallas as pl
from jax.experimental.pallas import tpu as pltpu
from jax.experimental.pallas import tpu_sc as plsc

_H = 32
_W = 32
_D = 384
_N_BOXES = 5000
_NSC = 3072
_NTC = _N_BOXES - _NSC
_TCB = 256
_NTC_PAD = -(-_NTC // _TCB) * _TCB
_NB = _NSC
_N_TILES = 32
_BPW = _NB // _N_TILES
_CH = 16
_NCH = _BPW // _CH
_LANES = 16
_SW = 40
_SROWS = (_H + 1) * _SW
_TAIL = 8


def _integral_kernel(feat_ref, s_ref, rc_ref):
    acc = feat_ref[0]
    rc_ref[0] = acc
    for i in range(1, _H):
        acc = acc + feat_ref[i]
        rc_ref[i] = acc
    s_ref[0] = jnp.zeros((_SW, _D), jnp.float32)
    cacc = jnp.zeros((_H, _D), jnp.float32)
    s_ref[1:_H + 1, 0, :] = cacc
    for w in range(_W):
        cacc = cacc + rc_ref[:, w, :]
        s_ref[1:_H + 1, w + 1, :] = cacc


def _integral_image(feat_map):
    return pl.pallas_call(
        _integral_kernel,
        out_shape=jax.ShapeDtypeStruct((_H + 1, _SW, _D), jnp.float32),
        scratch_shapes=[pltpu.VMEM((_H, _W, _D), jnp.float32)],
    )(feat_map)


def _round_half_even_nonneg(t):
    i = t.astype(jnp.int32)
    f = t - i.astype(jnp.float32)
    up = (f > 0.5) | ((f == 0.5) & ((i & 1) == 1))
    return jnp.where(up, i + 1, i)


def _sc_body(s_hbm, x1_hbm, y1_hbm, x2_hbm, y2_hbm, out_hbm,
             x1_v, y1_v, x2_v, y2_v, idx_a, idx_b, rows_a, rows_b,
             out_a, out_b, inv_v, gsem_a, gsem_b, osem_a, osem_b, bsem):
    wid = lax.axis_index("s") * 2 + lax.axis_index("c")
    base = wid * _BPW
    pltpu.async_copy(x1_hbm.at[pl.ds(base, _BPW)], x1_v, bsem)
    pltpu.async_copy(y1_hbm.at[pl.ds(base, _BPW)], y1_v, bsem)
    pltpu.async_copy(x2_hbm.at[pl.ds(base, _BPW)], x2_v, bsem)
    pltpu.async_copy(y2_hbm.at[pl.ds(base, _BPW)], y2_v, bsem)
    for v in (x1_v, y1_v, x2_v, y2_v):
        pltpu.make_async_copy(x1_hbm.at[pl.ds(base, _BPW)], v, bsem).wait()

    def fill_idx(ci, idx_r):
        off = ci * _CH
        x1 = x1_v[pl.ds(off, _LANES)]
        y1 = y1_v[pl.ds(off, _LANES)]
        x2 = x2_v[pl.ds(off, _LANES)]
        y2 = y2_v[pl.ds(off, _LANES)]
        zero = jnp.zeros((_LANES,), jnp.int32)
        wvec = jnp.full((_LANES,), _W, jnp.int32)
        hvec = jnp.full((_LANES,), _H, jnp.int32)
        cl = jnp.maximum(zero, (x1 * float(_W)).astype(jnp.int32))
        ch = jnp.minimum(wvec, jnp.maximum(
            cl + 1, _round_half_even_nonneg(x2 * float(_W) + 0.5)))
        rl = jnp.maximum(zero, (y1 * float(_H)).astype(jnp.int32))
        rh = jnp.minimum(hvec, jnp.maximum(
            rl + 1, _round_half_even_nonneg(y2 * float(_H) + 0.5)))
        stride = _SW
        idx_r[pl.ds(0, _LANES)] = rl * stride + cl
        idx_r[pl.ds(16, _LANES)] = rl * stride + ch
        idx_r[pl.ds(32, _LANES)] = rh * stride + cl
        idx_r[pl.ds(48, _LANES)] = rh * stride + ch
        cnt = (rh - rl) * (ch - cl)
        inv_v[pl.ds(off, _LANES)] = 1.0 / cnt.astype(jnp.float32)

    def combine(ci, rows_r, out_r):
        iv = inv_v[pl.ds(ci * _CH, _LANES)]

        def box(b, carry):
            ib = iv.at[jnp.full((_LANES,), b, jnp.int32)].get(
                mode="promise_in_bounds")
            for d in range(_D // _LANES):
                ds_ = pl.ds(d * _LANES, _LANES)
                out_r[b, ds_] = (rows_r[48 + b, ds_] - rows_r[16 + b, ds_]
                                 - rows_r[32 + b, ds_] + rows_r[b, ds_]) * ib
            return carry

        lax.fori_loop(0, _CH, box, None)

    fill_idx(0, idx_a)
    pltpu.async_copy(s_hbm.at[idx_a], rows_a, gsem_a)
    fill_idx(1, idx_b)
    pltpu.async_copy(s_hbm.at[idx_b], rows_b, gsem_b)

    def out_issue(ci, out_r, osem):
        start = base + ci * _CH
        full = start + _CH <= _NB
        part = (start < _NB) & (start + _CH > _NB)

        @pl.when(full)
        def _():
            pltpu.async_copy(out_r, out_hbm.at[pl.ds(start, _CH)], osem)

        @pl.when(part)
        def _():
            pltpu.async_copy(out_r.at[pl.ds(0, _TAIL)],
                             out_hbm.at[pl.ds(start, _TAIL)], osem)

    def out_wait(ci, out_r, osem):
        start = base + ci * _CH
        full = start + _CH <= _NB
        part = (start < _NB) & (start + _CH > _NB)

        @pl.when(full)
        def _():
            pltpu.make_async_copy(
                out_r, out_hbm.at[pl.ds(base, _CH)], osem).wait()

        @pl.when(part)
        def _():
            pltpu.make_async_copy(out_r.at[pl.ds(0, _TAIL)],
                                  out_hbm.at[pl.ds(base, _TAIL)], osem).wait()

    def stage(g, ci, idx_r, rows_r, out_r, gsem, osem):
        pltpu.make_async_copy(s_hbm.at[idx_r], rows_r, gsem).wait()

        @pl.when(g > 0)
        def _():
            out_wait(ci - 2, out_r, osem)

        combine(ci, rows_r, out_r)
        out_issue(ci, out_r, osem)

        @pl.when(ci + 2 < _NCH)
        def _():
            fill_idx(ci + 2, idx_r)
            pltpu.async_copy(s_hbm.at[idx_r], rows_r, gsem)

    def pair(g, carry):
        stage(g, 2 * g, idx_a, rows_a, out_a, gsem_a, osem_a)
        stage(g, 2 * g + 1, idx_b, rows_b, out_b, gsem_b, osem_b)
        return carry

    lax.fori_loop(0, _NCH // 2, pair, None)
    out_wait(_NCH - 2, out_a, osem_a)
    out_wait(_NCH - 1, out_b, osem_b)


def _roi_pool_sc(s_flat, x1, y1, x2, y2):
    mesh = plsc.VectorSubcoreMesh(core_axis_name="c", subcore_axis_name="s")
    f = functools.partial(
        pl.kernel,
        out_type=jax.ShapeDtypeStruct((_NB, _D), jnp.float32),
        mesh=mesh,
        scratch_types=[
            pltpu.VMEM((_BPW,), jnp.float32),
            pltpu.VMEM((_BPW,), jnp.float32),
            pltpu.VMEM((_BPW,), jnp.float32),
            pltpu.VMEM((_BPW,), jnp.float32),
            pltpu.VMEM((4 * _CH,), jnp.int32),
            pltpu.VMEM((4 * _CH,), jnp.int32),
            pltpu.VMEM((4 * _CH, _D), jnp.float32),
            pltpu.VMEM((4 * _CH, _D), jnp.float32),
            pltpu.VMEM((_CH, _D), jnp.float32),
            pltpu.VMEM((_CH, _D), jnp.float32),
            pltpu.VMEM((_BPW,), jnp.float32),
            pltpu.SemaphoreType.DMA,
            pltpu.SemaphoreType.DMA,
            pltpu.SemaphoreType.DMA,
            pltpu.SemaphoreType.DMA,
            pltpu.SemaphoreType.DMA,
        ],
    )(_sc_body)
    return f(s_flat, x1, y1, x2, y2)


def _tc_einsum_kernel(x1_ref, y1_ref, x2_ref, y2_ref, hi_ref, lo_ref, out_ref):
    x1 = x1_ref[...]
    y1 = y1_ref[...]
    x2 = x2_ref[...]
    y2 = y2_ref[...]
    cl = jnp.maximum(0, (x1 * float(_W)).astype(jnp.int32))
    ch = jnp.minimum(_W, jnp.maximum(
        cl + 1, _round_half_even_nonneg(x2 * float(_W) + 0.5)))
    rl = jnp.maximum(0, (y1 * float(_H)).astype(jnp.int32))
    rh = jnp.minimum(_H, jnp.maximum(
        rl + 1, _round_half_even_nonneg(y2 * float(_H) + 0.5)))
    one = jnp.float32(1.0)
    zero = jnp.float32(0.0)
    jv = lax.broadcasted_iota(jnp.int32, (_TCB, _H * _W), 1)
    jm = jv & (_W - 1)
    cond = ((jv >= (rl * _W)[:, None]) & (jv < (rh * _W)[:, None])
            & (jm >= cl[:, None]) & (jm < ch[:, None]))
    mask = jnp.where(cond, one, zero).astype(jnp.bfloat16)
    acc = (jnp.dot(mask, hi_ref[...], preferred_element_type=jnp.float32)
           + jnp.dot(mask, lo_ref[...], preferred_element_type=jnp.float32))
    cnt = ((rh - rl) * (ch - cl)).astype(jnp.float32)
    out_ref[...] = acc * (1.0 / cnt)[:, None]


def _roi_pool_tc(x1, y1, x2, y2, hi, lo):
    grid = _NTC_PAD // _TCB
    return pl.pallas_call(
        _tc_einsum_kernel,
        grid=(grid,),
        in_specs=[
            pl.BlockSpec((_TCB,), lambda i: (i,)),
            pl.BlockSpec((_TCB,), lambda i: (i,)),
            pl.BlockSpec((_TCB,), lambda i: (i,)),
            pl.BlockSpec((_TCB,), lambda i: (i,)),
            pl.BlockSpec((_H * _W, _D), lambda i: (0, 0)),
            pl.BlockSpec((_H * _W, _D), lambda i: (0, 0)),
        ],
        out_specs=pl.BlockSpec((_TCB, _D), lambda i: (i, 0)),
        out_shape=jax.ShapeDtypeStruct((_N_BOXES, _D), jnp.float32),
    )(x1, y1, x2, y2, hi, lo)


def kernel(feat_map, boxes):
    s_flat = _integral_image(feat_map).reshape(_SROWS, _D)
    bp = jnp.zeros((_NTC_PAD + _NSC, 4), jnp.float32).at[:_N_BOXES].set(boxes)
    feat2 = feat_map.reshape(_H * _W, _D)
    hi = feat2.astype(jnp.bfloat16)
    lo = (feat2 - hi.astype(jnp.float32)).astype(jnp.bfloat16)
    out_sc = _roi_pool_sc(s_flat, bp[_NTC:_NTC + _NSC, 0],
                          bp[_NTC:_NTC + _NSC, 1],
                          bp[_NTC:_NTC + _NSC, 2],
                          bp[_NTC:_NTC + _NSC, 3])
    out_tc = _roi_pool_tc(bp[:_NTC_PAD, 0], bp[:_NTC_PAD, 1],
                          bp[:_NTC_PAD, 2], bp[:_NTC_PAD, 3], hi, lo)
    return lax.dynamic_update_slice(out_tc, out_sc, (_NTC, 0))
